# baseline (speedup 1.0000x reference)
import functools

import jax
import jax.numpy as jnp
from jax import lax
from jax.experimental import pallas as pl
from jax.experimental.pallas import tpu as pltpu
from jax.experimental.pallas import tpu_sc as plsc

_NC = 2
_NS = 16
_NW = _NC * _NS

_LANES = 16
_UNROLL = 8


def _sc_gather_t_body(idxT_hbm, tableT_hbm, out_hbm, col_v, idx_v, out_v,
                      sem_i0, sem_i1, sem_o0, sem_o1):
  h_n = idxT_hbm.shape[0]
  b = idxT_hbm.shape[1]
  d_n = tableT_hbm.shape[0]
  n_pass = d_n // _NW

  wid = lax.axis_index("s") * _NC + lax.axis_index("c")
  sem_i = (sem_i0, sem_i1)
  sem_o = (sem_o0, sem_o1)

  pltpu.async_copy(idxT_hbm.at[0], idx_v.at[0], sem_i[0])
  pltpu.async_copy(idxT_hbm.at[1], idx_v.at[1], sem_i[1])

  n_chunk = b // (_LANES * _UNROLL)

  for p in range(n_pass):
    d = p * _NW + wid
    pltpu.sync_copy(tableT_hbm.at[d], col_v)

    def pair(i, carry, p=p, d=d):
      for sb in range(2):
        h = 2 * i + sb

        pltpu.make_async_copy(idxT_hbm.at[0], idx_v.at[sb], sem_i[sb]).wait()

        if p == 0:
          @pl.when(i > 0)
          def _(sb=sb):
            pltpu.make_async_copy(
                out_v.at[sb], out_hbm.at[0], sem_o[sb]).wait()
        else:
          pltpu.make_async_copy(out_v.at[sb], out_hbm.at[0], sem_o[sb]).wait()

        def chunk(j, c, sb=sb):
          base = j * (_LANES * _UNROLL)
          for u in range(_UNROLL):
            off = base + u * _LANES
            iv = idx_v.at[sb][pl.ds(off, _LANES)]
            out_v.at[sb][pl.ds(off, _LANES)] = plsc.load_gather(col_v, [iv])
          return c

        lax.fori_loop(0, n_chunk, chunk, 0)

        pltpu.async_copy(out_v.at[sb], out_hbm.at[h * d_n + d], sem_o[sb])

        nh = h + 2
        nh = jnp.where(nh >= h_n, nh - h_n, nh)
        pltpu.async_copy(idxT_hbm.at[nh], idx_v.at[sb], sem_i[sb])
      return carry

    lax.fori_loop(0, h_n // 2, pair, 0)

  for sb in range(2):
    pltpu.make_async_copy(out_v.at[sb], out_hbm.at[0], sem_o[sb]).wait()
    pltpu.make_async_copy(idxT_hbm.at[0], idx_v.at[sb], sem_i[sb]).wait()


def _sc_gather_t(idxT, tableT):
  h_n, b = idxT.shape
  d_n, v = tableT.shape
  mesh = plsc.VectorSubcoreMesh(core_axis_name="c", subcore_axis_name="s")
  f = functools.partial(
      pl.kernel,
      out_type=jax.ShapeDtypeStruct((h_n * d_n, b), jnp.float32),
      mesh=mesh,
      scratch_types=[
          pltpu.VMEM((v,), jnp.float32),
          pltpu.VMEM((2, b), jnp.int32),
          pltpu.VMEM((2, b), jnp.float32),
          pltpu.SemaphoreType.DMA,
          pltpu.SemaphoreType.DMA,
          pltpu.SemaphoreType.DMA,
          pltpu.SemaphoreType.DMA,
      ],
      compiler_params=pltpu.CompilerParams(
          use_tc_tiling_on_sc=False, needs_layout_passes=False),
  )(_sc_gather_t_body)
  return f(idxT, tableT)


def _transpose_body(x_ref, y_ref):
  y_ref[...] = x_ref[...].T


def _tc_transpose(x, rb):
  n, m = x.shape
  grid = (n // rb,)
  return pl.pallas_call(
      _transpose_body,
      grid=grid,
      in_specs=[pl.BlockSpec((rb, m), lambda i: (i, 0))],
      out_specs=pl.BlockSpec((m, rb), lambda i: (0, i)),
      out_shape=jax.ShapeDtypeStruct((m, n), x.dtype),
  )(x)


def _masklen_body(x_ref, mask_ref, len_ref):
  x = x_ref[...]
  m = x != 0
  mask_ref[...] = m
  len_ref[...] = jnp.sum(m.astype(jnp.int32), axis=1)


def _tc_masklen(inputs):
  b, h = inputs.shape
  rb = 512
  grid = (b // rb,)
  return pl.pallas_call(
      _masklen_body,
      grid=grid,
      in_specs=[pl.BlockSpec((rb, h), lambda i: (i, 0))],
      out_specs=[
          pl.BlockSpec((rb, h), lambda i: (i, 0)),
          pl.BlockSpec((rb,), lambda i: (i,)),
      ],
      out_shape=[
          jax.ShapeDtypeStruct((b, h), jnp.bool_),
          jax.ShapeDtypeStruct((b,), jnp.int32),
      ],
  )(inputs)


@jax.jit
def kernel(inputs, emb_table):
  b, h = inputs.shape
  v, d = emb_table.shape
  idxT = _tc_transpose(inputs, 512)
  vp = (v + 2047) // 2048 * 2048
  table_p = jnp.pad(emb_table, ((0, vp - v), (0, 0)))
  tableT = _tc_transpose(table_p, 2048)
  out2d = _sc_gather_t(idxT, tableT)
  masks, lengths = _tc_masklen(inputs)
  emb = jnp.transpose(out2d.reshape(h, d, b), (2, 0, 1))
  return emb, lengths, masks


# speedup vs baseline: 1.2415x; 1.2415x over previous
import functools

import jax
import jax.numpy as jnp
from jax import lax
from jax.experimental import pallas as pl
from jax.experimental.pallas import tpu as pltpu
from jax.experimental.pallas import tpu_sc as plsc

_NC = 2
_NS = 16
_NW = _NC * _NS

_LANES = 16
_UNROLL = 8


def _sc_gather_t_body(idxT_hbm, tableT_hbm, out_hbm, col_v, idx_v, out_v,
                      sem_i0, sem_i1, sem_o0, sem_o1):
  h_n = idxT_hbm.shape[0]
  b = idxT_hbm.shape[1]
  d_n = tableT_hbm.shape[0]
  n_pass = d_n // _NW

  wid = lax.axis_index("s") * _NC + lax.axis_index("c")
  sem_i = (sem_i0, sem_i1)
  sem_o = (sem_o0, sem_o1)

  pltpu.async_copy(idxT_hbm.at[0], idx_v.at[0], sem_i[0])
  pltpu.async_copy(idxT_hbm.at[1], idx_v.at[1], sem_i[1])

  n_chunk = b // (_LANES * _UNROLL)

  for p in range(n_pass):
    d = p * _NW + wid
    pltpu.sync_copy(tableT_hbm.at[d], col_v)

    def pair(i, carry, p=p, d=d):
      for sb in range(2):
        h = 2 * i + sb

        pltpu.make_async_copy(idxT_hbm.at[0], idx_v.at[sb], sem_i[sb]).wait()

        if p == 0:
          @pl.when(i > 0)
          def _(sb=sb):
            pltpu.make_async_copy(
                out_v.at[sb], out_hbm.at[0], sem_o[sb]).wait()
        else:
          pltpu.make_async_copy(out_v.at[sb], out_hbm.at[0], sem_o[sb]).wait()

        def chunk(j, c, sb=sb):
          base = j * (_LANES * _UNROLL)
          ivs = [idx_v.at[sb][pl.ds(base + u * _LANES, _LANES)]
                 for u in range(_UNROLL)]
          vals = [plsc.load_gather(col_v, [iv]) for iv in ivs]
          for u in range(_UNROLL):
            out_v.at[sb][pl.ds(base + u * _LANES, _LANES)] = vals[u]
          return c

        lax.fori_loop(0, n_chunk, chunk, 0)

        pltpu.async_copy(out_v.at[sb], out_hbm.at[h * d_n + d], sem_o[sb])

        nh = h + 2
        nh = jnp.where(nh >= h_n, nh - h_n, nh)
        pltpu.async_copy(idxT_hbm.at[nh], idx_v.at[sb], sem_i[sb])
      return carry

    lax.fori_loop(0, h_n // 2, pair, 0)

  for sb in range(2):
    pltpu.make_async_copy(out_v.at[sb], out_hbm.at[0], sem_o[sb]).wait()
    pltpu.make_async_copy(idxT_hbm.at[0], idx_v.at[sb], sem_i[sb]).wait()


def _sc_gather_t(idxT, tableT):
  h_n, b = idxT.shape
  d_n, v = tableT.shape
  mesh = plsc.VectorSubcoreMesh(core_axis_name="c", subcore_axis_name="s")
  f = functools.partial(
      pl.kernel,
      out_type=jax.ShapeDtypeStruct((h_n * d_n, b), jnp.float32),
      mesh=mesh,
      scratch_types=[
          pltpu.VMEM((v,), jnp.float32),
          pltpu.VMEM((2, b), jnp.int32),
          pltpu.VMEM((2, b), jnp.float32),
          pltpu.SemaphoreType.DMA,
          pltpu.SemaphoreType.DMA,
          pltpu.SemaphoreType.DMA,
          pltpu.SemaphoreType.DMA,
      ],
      compiler_params=pltpu.CompilerParams(
          use_tc_tiling_on_sc=False, needs_layout_passes=False),
  )(_sc_gather_t_body)
  return f(idxT, tableT)


def _transpose_body(x_ref, y_ref):
  y_ref[...] = x_ref[...].T


def _tc_transpose(x, rb):
  n, m = x.shape
  grid = (n // rb,)
  return pl.pallas_call(
      _transpose_body,
      grid=grid,
      in_specs=[pl.BlockSpec((rb, m), lambda i: (i, 0))],
      out_specs=pl.BlockSpec((m, rb), lambda i: (0, i)),
      out_shape=jax.ShapeDtypeStruct((m, n), x.dtype),
  )(x)


def _masklen_body(x_ref, mask_ref, len_ref):
  x = x_ref[...]
  m = x != 0
  mask_ref[...] = m
  len_ref[...] = jnp.sum(m.astype(jnp.int32), axis=1)


def _tc_masklen(inputs):
  b, h = inputs.shape
  rb = 512
  grid = (b // rb,)
  return pl.pallas_call(
      _masklen_body,
      grid=grid,
      in_specs=[pl.BlockSpec((rb, h), lambda i: (i, 0))],
      out_specs=[
          pl.BlockSpec((rb, h), lambda i: (i, 0)),
          pl.BlockSpec((rb,), lambda i: (i,)),
      ],
      out_shape=[
          jax.ShapeDtypeStruct((b, h), jnp.bool_),
          jax.ShapeDtypeStruct((b,), jnp.int32),
      ],
  )(inputs)


@jax.jit
def kernel(inputs, emb_table):
  b, h = inputs.shape
  v, d = emb_table.shape
  idxT = _tc_transpose(inputs, 512)
  vp = (v + 2047) // 2048 * 2048
  table_p = jnp.pad(emb_table, ((0, vp - v), (0, 0)))
  tableT = _tc_transpose(table_p, 2048)
  out2d = _sc_gather_t(idxT, tableT)
  masks, lengths = _tc_masklen(inputs)
  emb = jnp.transpose(out2d.reshape(h, d, b), (2, 0, 1))
  return emb, lengths, masks


# speedup vs baseline: 1.2432x; 1.0013x over previous
import functools

import jax
import jax.numpy as jnp
from jax import lax
from jax.experimental import pallas as pl
from jax.experimental.pallas import tpu as pltpu
from jax.experimental.pallas import tpu_sc as plsc

_NC = 2
_NS = 16
_NW = _NC * _NS

_LANES = 16
_UNROLL = 16


def _sc_gather_t_body(idxT_hbm, tableT_hbm, out_hbm, col_v, idx_v, out_v,
                      sem_i0, sem_i1, sem_o0, sem_o1):
  h_n = idxT_hbm.shape[0]
  b = idxT_hbm.shape[1]
  d_n = tableT_hbm.shape[0]
  n_pass = d_n // _NW

  wid = lax.axis_index("s") * _NC + lax.axis_index("c")
  sem_i = (sem_i0, sem_i1)
  sem_o = (sem_o0, sem_o1)

  pltpu.async_copy(idxT_hbm.at[0], idx_v.at[0], sem_i[0])
  pltpu.async_copy(idxT_hbm.at[1], idx_v.at[1], sem_i[1])

  n_chunk = b // (_LANES * _UNROLL)

  for p in range(n_pass):
    d = p * _NW + wid
    pltpu.sync_copy(tableT_hbm.at[d], col_v)

    def pair(i, carry, p=p, d=d):
      for sb in range(2):
        h = 2 * i + sb

        pltpu.make_async_copy(idxT_hbm.at[0], idx_v.at[sb], sem_i[sb]).wait()

        if p == 0:
          @pl.when(i > 0)
          def _(sb=sb):
            pltpu.make_async_copy(
                out_v.at[sb], out_hbm.at[0], sem_o[sb]).wait()
        else:
          pltpu.make_async_copy(out_v.at[sb], out_hbm.at[0], sem_o[sb]).wait()

        def chunk(j, c, sb=sb):
          base = j * (_LANES * _UNROLL)
          ivs = [idx_v.at[sb][pl.ds(base + u * _LANES, _LANES)]
                 for u in range(_UNROLL)]
          vals = [plsc.load_gather(col_v, [iv]) for iv in ivs]
          for u in range(_UNROLL):
            out_v.at[sb][pl.ds(base + u * _LANES, _LANES)] = vals[u]
          return c

        lax.fori_loop(0, n_chunk, chunk, 0)

        pltpu.async_copy(out_v.at[sb], out_hbm.at[h * d_n + d], sem_o[sb])

        nh = h + 2
        nh = jnp.where(nh >= h_n, nh - h_n, nh)
        pltpu.async_copy(idxT_hbm.at[nh], idx_v.at[sb], sem_i[sb])
      return carry

    lax.fori_loop(0, h_n // 2, pair, 0)

  for sb in range(2):
    pltpu.make_async_copy(out_v.at[sb], out_hbm.at[0], sem_o[sb]).wait()
    pltpu.make_async_copy(idxT_hbm.at[0], idx_v.at[sb], sem_i[sb]).wait()


def _sc_gather_t(idxT, tableT):
  h_n, b = idxT.shape
  d_n, v = tableT.shape
  mesh = plsc.VectorSubcoreMesh(core_axis_name="c", subcore_axis_name="s")
  f = functools.partial(
      pl.kernel,
      out_type=jax.ShapeDtypeStruct((h_n * d_n, b), jnp.float32),
      mesh=mesh,
      scratch_types=[
          pltpu.VMEM((v,), jnp.float32),
          pltpu.VMEM((2, b), jnp.int32),
          pltpu.VMEM((2, b), jnp.float32),
          pltpu.SemaphoreType.DMA,
          pltpu.SemaphoreType.DMA,
          pltpu.SemaphoreType.DMA,
          pltpu.SemaphoreType.DMA,
      ],
      compiler_params=pltpu.CompilerParams(
          use_tc_tiling_on_sc=False, needs_layout_passes=False),
  )(_sc_gather_t_body)
  return f(idxT, tableT)


def _transpose_body(x_ref, y_ref):
  y_ref[...] = x_ref[...].T


def _tc_transpose(x, rb):
  n, m = x.shape
  grid = (n // rb,)
  return pl.pallas_call(
      _transpose_body,
      grid=grid,
      in_specs=[pl.BlockSpec((rb, m), lambda i: (i, 0))],
      out_specs=pl.BlockSpec((m, rb), lambda i: (0, i)),
      out_shape=jax.ShapeDtypeStruct((m, n), x.dtype),
  )(x)


def _masklen_body(x_ref, mask_ref, len_ref):
  x = x_ref[...]
  m = x != 0
  mask_ref[...] = m
  len_ref[...] = jnp.sum(m.astype(jnp.int32), axis=1)


def _tc_masklen(inputs):
  b, h = inputs.shape
  rb = 512
  grid = (b // rb,)
  return pl.pallas_call(
      _masklen_body,
      grid=grid,
      in_specs=[pl.BlockSpec((rb, h), lambda i: (i, 0))],
      out_specs=[
          pl.BlockSpec((rb, h), lambda i: (i, 0)),
          pl.BlockSpec((rb,), lambda i: (i,)),
      ],
      out_shape=[
          jax.ShapeDtypeStruct((b, h), jnp.bool_),
          jax.ShapeDtypeStruct((b,), jnp.int32),
      ],
  )(inputs)


@jax.jit
def kernel(inputs, emb_table):
  b, h = inputs.shape
  v, d = emb_table.shape
  idxT = _tc_transpose(inputs, 512)
  vp = (v + 2047) // 2048 * 2048
  table_p = jnp.pad(emb_table, ((0, vp - v), (0, 0)))
  tableT = _tc_transpose(table_p, 2048)
  out2d = _sc_gather_t(idxT, tableT)
  masks, lengths = _tc_masklen(inputs)
  emb = jnp.transpose(out2d.reshape(h, d, b), (2, 0, 1))
  return emb, lengths, masks


# speedup vs baseline: 1.8148x; 1.4598x over previous
import functools

import jax
import jax.numpy as jnp
from jax import lax
from jax.experimental import pallas as pl
from jax.experimental.pallas import tpu as pltpu
from jax.experimental.pallas import tpu_sc as plsc

_NC = 2
_NS = 16
_NW = _NC * _NS

_LANES = 16
_UNROLL = 16


def _sc_gather_t_body(idxT_hbm, tableT_hbm, out_hbm, col_v, idx_v, out_v,
                      sem_i0, sem_i1, sem_o0, sem_o1):
  h_n = idxT_hbm.shape[0]
  b = idxT_hbm.shape[1]
  d_n = tableT_hbm.shape[0]
  n_pass = d_n // _NW

  wid = lax.axis_index("s") * _NC + lax.axis_index("c")
  sem_i = (sem_i0, sem_i1)
  sem_o = (sem_o0, sem_o1)
  n_tc = b // 128

  pltpu.async_copy(idxT_hbm.at[0], idx_v.at[0], sem_i[0])
  pltpu.async_copy(idxT_hbm.at[1], idx_v.at[1], sem_i[1])

  n_chunk = b // (_LANES * _UNROLL)

  for p in range(n_pass):
    d = p * _NW + wid
    tr = d // 8
    r = d % 8
    pltpu.sync_copy(tableT_hbm.at[d], col_v)

    def pair(i, carry, p=p, tr=tr, r=r):
      for sb in range(2):
        h = 2 * i + sb

        pltpu.make_async_copy(idxT_hbm.at[0], idx_v.at[sb], sem_i[sb]).wait()

        if p == 0:
          @pl.when(i > 0)
          def _(sb=sb):
            pltpu.make_async_copy(
                out_v.at[sb], out_hbm.at[pl.ds(0, n_tc), 0], sem_o[sb]).wait()
        else:
          pltpu.make_async_copy(out_v.at[sb], out_hbm.at[pl.ds(0, n_tc), 0], sem_o[sb]).wait()

        def chunk(j, c, sb=sb):
          base = j * (_LANES * _UNROLL)
          ivs = [idx_v.at[sb][pl.ds(base + u * _LANES, _LANES)]
                 for u in range(_UNROLL)]
          vals = [plsc.load_gather(col_v, [iv]) for iv in ivs]
          for u in range(_UNROLL):
            row = j * (_LANES * _UNROLL // 128) + u // 8
            out_v.at[sb].at[row][pl.ds((u % 8) * _LANES, _LANES)] = vals[u]
          return c

        lax.fori_loop(0, n_chunk, chunk, 0)

        g0 = (h * (d_n // 8) + tr) * n_tc
        pltpu.async_copy(out_v.at[sb], out_hbm.at[pl.ds(g0, n_tc), r],
                         sem_o[sb])

        nh = h + 2
        nh = jnp.where(nh >= h_n, nh - h_n, nh)
        pltpu.async_copy(idxT_hbm.at[nh], idx_v.at[sb], sem_i[sb])
      return carry

    lax.fori_loop(0, h_n // 2, pair, 0)

  for sb in range(2):
    pltpu.make_async_copy(out_v.at[sb], out_hbm.at[pl.ds(0, n_tc), 0], sem_o[sb]).wait()
    pltpu.make_async_copy(idxT_hbm.at[0], idx_v.at[sb], sem_i[sb]).wait()


def _sc_gather_t(idxT, tableT):
  h_n, b = idxT.shape
  d_n, v = tableT.shape
  mesh = plsc.VectorSubcoreMesh(core_axis_name="c", subcore_axis_name="s")
  f = functools.partial(
      pl.kernel,
      out_type=jax.ShapeDtypeStruct((h_n * d_n * b // 1024, 8, 128),
                                    jnp.float32),
      mesh=mesh,
      scratch_types=[
          pltpu.VMEM((v,), jnp.float32),
          pltpu.VMEM((2, b), jnp.int32),
          pltpu.VMEM((2, b // 128, 128), jnp.float32),
          pltpu.SemaphoreType.DMA,
          pltpu.SemaphoreType.DMA,
          pltpu.SemaphoreType.DMA,
          pltpu.SemaphoreType.DMA,
      ],
      compiler_params=pltpu.CompilerParams(
          use_tc_tiling_on_sc=False, needs_layout_passes=False),
  )(_sc_gather_t_body)
  return f(idxT, tableT)


def _transpose_body(x_ref, y_ref):
  y_ref[...] = x_ref[...].T


def _tc_transpose(x, rb):
  n, m = x.shape
  grid = (n // rb,)
  return pl.pallas_call(
      _transpose_body,
      grid=grid,
      in_specs=[pl.BlockSpec((rb, m), lambda i: (i, 0))],
      out_specs=pl.BlockSpec((m, rb), lambda i: (0, i)),
      out_shape=jax.ShapeDtypeStruct((m, n), x.dtype),
  )(x)


def _masklen_body(x_ref, mask_ref, len_ref):
  x = x_ref[...]
  m = x != 0
  mask_ref[...] = m
  len_ref[...] = jnp.sum(m.astype(jnp.int32), axis=1)


def _tc_masklen(inputs):
  b, h = inputs.shape
  rb = 512
  grid = (b // rb,)
  return pl.pallas_call(
      _masklen_body,
      grid=grid,
      in_specs=[pl.BlockSpec((rb, h), lambda i: (i, 0))],
      out_specs=[
          pl.BlockSpec((rb, h), lambda i: (i, 0)),
          pl.BlockSpec((rb,), lambda i: (i,)),
      ],
      out_shape=[
          jax.ShapeDtypeStruct((b, h), jnp.bool_),
          jax.ShapeDtypeStruct((b,), jnp.int32),
      ],
  )(inputs)


@jax.jit
def kernel(inputs, emb_table):
  b, h = inputs.shape
  v, d = emb_table.shape
  idxT = _tc_transpose(inputs, 512)
  vp = (v + 2047) // 2048 * 2048
  table_p = jnp.pad(emb_table, ((0, vp - v), (0, 0)))
  tableT = _tc_transpose(table_p, 2048)
  out3 = _sc_gather_t(idxT, tableT)
  masks, lengths = _tc_masklen(inputs)
  out5 = out3.reshape(h, d // 8, b // 128, 8, 128)
  emb = jnp.transpose(out5, (2, 4, 0, 1, 3)).reshape(b, h, d)
  return emb, lengths, masks
